# baseline (speedup 1.0000x reference)
import functools

import jax
import jax.numpy as jnp
from jax.experimental import pallas as pl
from jax.experimental.pallas import tpu as pltpu

BLK = 256
LEAK = 0.2


def _attention_factors(es, ed, dtype):
    ces = jnp.max(es, axis=0, keepdims=True)
    ced = jnp.max(ed, axis=0, keepdims=True)
    f1 = jnp.exp(es - ces)
    f2k = jnp.exp(LEAK * (es - ces) - (1.0 - LEAK) * (ces + ced))
    g1 = jnp.exp(ed - ced)
    g2 = jnp.exp(LEAK * (ed - ced))
    c = lambda v: v.astype(dtype)
    return c(f1), c(f2k), c(g1.T), c(g2.T)


def _block_weights(adj_blk, i, head, f1_ref, f2k_ref, g1t_ref, g2t_ref):
    rows = pl.ds(i * BLK, BLK)
    w1 = f1_ref[rows, head : head + 1] * g1t_ref[head : head + 1, :]
    w2 = f2k_ref[rows, head : head + 1] * g2t_ref[head : head + 1, :]
    return jnp.maximum(w1, w2) * adj_blk


def _fused_kernel(x_ref, adj_ref, w1_ref, asrc1_ref, adst1_ref, w2_ref,
                  asrc2_ref, adst2_ref, out_ref,
                  adj16_scr, h1_scr, hx1_scr, f11_scr, f21_scr, g11_scr,
                  g21_scr, hx2_scr, f12_scr, f22_scr, g12_scr, g22_scr,
                  *, heads, out1):
    ph = pl.program_id(0)
    i = pl.program_id(1)
    ext = out1 + 1
    classes = hx2_scr.shape[1] - 1

    @pl.when((ph == 0) & (i == 0))
    def _precompute1():
        h = jnp.dot(x_ref[...], w1_ref[...], preferred_element_type=jnp.float32)
        es = jnp.dot(h, asrc1_ref[...], preferred_element_type=jnp.float32)
        ed = jnp.dot(h, adst1_ref[...], preferred_element_type=jnp.float32)
        f1, f2k, g1t, g2t = _attention_factors(es, ed, jnp.bfloat16)
        f11_scr[...] = f1
        f21_scr[...] = f2k
        g11_scr[...] = g1t
        g21_scr[...] = g2t
        h16 = h.astype(jnp.bfloat16)
        for head in range(heads):
            hx1_scr[:, head * ext:head * ext + out1] = (
                h16[:, head * out1:(head + 1) * out1])
            hx1_scr[:, head * ext + out1:(head + 1) * ext] = jnp.ones(
                (h.shape[0], 1), jnp.bfloat16)

    @pl.when(ph == 0)
    def _layer1_block():
        adj_blk = adj_ref[...].astype(jnp.bfloat16)
        adj16_scr[pl.ds(i * BLK, BLK), :] = adj_blk
        for head in range(heads):
            p = _block_weights(adj_blk, i, head, f11_scr, f21_scr,
                               g11_scr, g21_scr)
            ne = jnp.dot(p, hx1_scr[:, head * ext:(head + 1) * ext],
                         preferred_element_type=jnp.float32)
            o = ne[:, :out1] / ne[:, out1:]
            h1_scr[pl.ds(i * BLK, BLK), head * out1:(head + 1) * out1] = (
                jnp.where(o > 0, o, jnp.exp(o) - 1.0))

    @pl.when((ph == 1) & (i == 0))
    def _precompute2():
        h = jnp.dot(h1_scr[...], w2_ref[...],
                    preferred_element_type=jnp.float32)
        es = jnp.dot(h, asrc2_ref[...], preferred_element_type=jnp.float32)
        ed = jnp.dot(h, adst2_ref[...], preferred_element_type=jnp.float32)
        f1, f2k, g1t, g2t = _attention_factors(es, ed, jnp.bfloat16)
        f12_scr[...] = f1
        f22_scr[...] = f2k
        g12_scr[...] = g1t
        g22_scr[...] = g2t
        hx2_scr[:, :classes] = h.astype(jnp.bfloat16)
        hx2_scr[:, classes:] = jnp.ones((h.shape[0], 1), jnp.bfloat16)

    @pl.when(ph == 1)
    def _layer2_block():
        adj_blk = adj16_scr[pl.ds(i * BLK, BLK), :]
        p = _block_weights(adj_blk, i, 0, f12_scr, f22_scr,
                           g12_scr, g22_scr)
        ne = jnp.dot(p, hx2_scr[...], preferred_element_type=jnp.float32)
        o = ne[:, :classes] / ne[:, classes:]
        m = jnp.max(o, axis=1, keepdims=True)
        lse = jnp.log(jnp.sum(jnp.exp(o - m), axis=1, keepdims=True)) + m
        out_ref[...] = o - lse


def kernel(x, adj, W1, a1_src, a1_dst, W2, a2_src, a2_dst):
    n, ins = x.shape
    heads, _, out1 = W1.shape
    classes = W2.shape[2]
    hidden = heads * out1

    w1f = jnp.transpose(W1, (1, 0, 2)).reshape(ins, hidden)
    eye = jnp.eye(heads, dtype=x.dtype)
    asrc1 = (a1_src[:, :, None] * eye[:, None, :]).reshape(hidden, heads)
    adst1 = (a1_dst[:, :, None] * eye[:, None, :]).reshape(hidden, heads)

    pad = jnp.zeros((classes, 7), x.dtype)
    a2p_src = jnp.concatenate([a2_src.reshape(classes, 1), pad], axis=1)
    a2p_dst = jnp.concatenate([a2_dst.reshape(classes, 1), pad], axis=1)

    full = lambda *dims: pl.BlockSpec(dims, lambda p, i: (0,) * len(dims))
    f32_scr = lambda r, c: pltpu.VMEM((r, c), jnp.float32)
    bf16_scr = lambda r, c: pltpu.VMEM((r, c), jnp.bfloat16)

    return pl.pallas_call(
        functools.partial(_fused_kernel, heads=heads, out1=out1),
        grid=(2, n // BLK),
        in_specs=[
            full(n, ins),
            pl.BlockSpec((BLK, n),
                         lambda p, i: (jnp.where(p == 0, i, n // BLK - 1), 0)),
            full(ins, hidden),
            full(hidden, heads),
            full(hidden, heads),
            full(hidden, classes),
            full(classes, 8),
            full(classes, 8),
        ],
        out_specs=pl.BlockSpec((BLK, classes), lambda p, i: (i, 0)),
        out_shape=jax.ShapeDtypeStruct((n, classes), jnp.float32),
        scratch_shapes=[
            bf16_scr(n, n),
            f32_scr(n, hidden),
            bf16_scr(n, (out1 + 1) * heads),
            bf16_scr(n, heads),
            bf16_scr(n, heads),
            bf16_scr(heads, n),
            bf16_scr(heads, n),
            bf16_scr(n, classes + 1),
            bf16_scr(n, 8),
            bf16_scr(n, 8),
            bf16_scr(8, n),
            bf16_scr(8, n),
        ],
    )(x, adj, w1f, asrc1, adst1, W2[0], a2p_src, a2p_dst)


# speedup vs baseline: 1.0829x; 1.0829x over previous
import functools

import jax
import jax.numpy as jnp
from jax.experimental import pallas as pl
from jax.experimental.pallas import tpu as pltpu

BLK = 512
LEAK = 0.2


def _attention_factors(es, ed, dtype):
    ces = jnp.max(es, axis=0, keepdims=True)
    ced = jnp.max(ed, axis=0, keepdims=True)
    f1 = jnp.exp(es - ces)
    f2k = jnp.exp(LEAK * (es - ces) - (1.0 - LEAK) * (ces + ced))
    g1 = jnp.exp(ed - ced)
    g2 = jnp.exp(LEAK * (ed - ced))
    c = lambda v: v.astype(dtype)
    return c(f1), c(f2k), c(g1.T), c(g2.T)


def _block_weights(adj_blk, i, head, f1_ref, f2k_ref, g1t_ref, g2t_ref):
    rows = pl.ds(i * BLK, BLK)
    w1 = f1_ref[rows, head : head + 1] * g1t_ref[head : head + 1, :]
    w2 = f2k_ref[rows, head : head + 1] * g2t_ref[head : head + 1, :]
    return jnp.maximum(w1, w2) * adj_blk


def _fused_kernel(x_ref, adj_ref, w1_ref, asrc1_ref, adst1_ref, w2_ref,
                  asrc2_ref, adst2_ref, out_ref,
                  adj16_scr, h1_scr, hx1_scr, f11_scr, f21_scr, g11_scr,
                  g21_scr, hx2_scr, f12_scr, f22_scr, g12_scr, g22_scr,
                  *, heads, out1):
    ph = pl.program_id(0)
    i = pl.program_id(1)
    ext = out1 + 1
    classes = hx2_scr.shape[1] - 1

    @pl.when((ph == 0) & (i == 0))
    def _precompute1():
        h = jnp.dot(x_ref[...], w1_ref[...], preferred_element_type=jnp.float32)
        es = jnp.dot(h, asrc1_ref[...], preferred_element_type=jnp.float32)
        ed = jnp.dot(h, adst1_ref[...], preferred_element_type=jnp.float32)
        f1, f2k, g1t, g2t = _attention_factors(es, ed, jnp.bfloat16)
        f11_scr[...] = f1
        f21_scr[...] = f2k
        g11_scr[...] = g1t
        g21_scr[...] = g2t
        h16 = h.astype(jnp.bfloat16)
        for head in range(heads):
            hx1_scr[:, head * ext:head * ext + out1] = (
                h16[:, head * out1:(head + 1) * out1])
            hx1_scr[:, head * ext + out1:(head + 1) * ext] = jnp.ones(
                (h.shape[0], 1), jnp.bfloat16)

    @pl.when(ph == 0)
    def _layer1_block():
        adj_blk = adj_ref[...].astype(jnp.bfloat16)
        adj16_scr[pl.ds(i * BLK, BLK), :] = adj_blk
        for head in range(heads):
            p = _block_weights(adj_blk, i, head, f11_scr, f21_scr,
                               g11_scr, g21_scr)
            ne = jnp.dot(p, hx1_scr[:, head * ext:(head + 1) * ext],
                         preferred_element_type=jnp.float32)
            o = ne[:, :out1] / ne[:, out1:]
            h1_scr[pl.ds(i * BLK, BLK), head * out1:(head + 1) * out1] = (
                jnp.where(o > 0, o, jnp.exp(o) - 1.0))

    @pl.when((ph == 1) & (i == 0))
    def _precompute2():
        h = jnp.dot(h1_scr[...], w2_ref[...],
                    preferred_element_type=jnp.float32)
        es = jnp.dot(h, asrc2_ref[...], preferred_element_type=jnp.float32)
        ed = jnp.dot(h, adst2_ref[...], preferred_element_type=jnp.float32)
        f1, f2k, g1t, g2t = _attention_factors(es, ed, jnp.bfloat16)
        f12_scr[...] = f1
        f22_scr[...] = f2k
        g12_scr[...] = g1t
        g22_scr[...] = g2t
        hx2_scr[:, :classes] = h.astype(jnp.bfloat16)
        hx2_scr[:, classes:] = jnp.ones((h.shape[0], 1), jnp.bfloat16)

    @pl.when(ph == 1)
    def _layer2_block():
        adj_blk = adj16_scr[pl.ds(i * BLK, BLK), :]
        p = _block_weights(adj_blk, i, 0, f12_scr, f22_scr,
                           g12_scr, g22_scr)
        ne = jnp.dot(p, hx2_scr[...], preferred_element_type=jnp.float32)
        o = ne[:, :classes] / ne[:, classes:]
        m = jnp.max(o, axis=1, keepdims=True)
        lse = jnp.log(jnp.sum(jnp.exp(o - m), axis=1, keepdims=True)) + m
        out_ref[...] = o - lse


def kernel(x, adj, W1, a1_src, a1_dst, W2, a2_src, a2_dst):
    n, ins = x.shape
    heads, _, out1 = W1.shape
    classes = W2.shape[2]
    hidden = heads * out1

    w1f = jnp.transpose(W1, (1, 0, 2)).reshape(ins, hidden)
    eye = jnp.eye(heads, dtype=x.dtype)
    asrc1 = (a1_src[:, :, None] * eye[:, None, :]).reshape(hidden, heads)
    adst1 = (a1_dst[:, :, None] * eye[:, None, :]).reshape(hidden, heads)

    pad = jnp.zeros((classes, 7), x.dtype)
    a2p_src = jnp.concatenate([a2_src.reshape(classes, 1), pad], axis=1)
    a2p_dst = jnp.concatenate([a2_dst.reshape(classes, 1), pad], axis=1)

    full = lambda *dims: pl.BlockSpec(dims, lambda p, i: (0,) * len(dims))
    f32_scr = lambda r, c: pltpu.VMEM((r, c), jnp.float32)
    bf16_scr = lambda r, c: pltpu.VMEM((r, c), jnp.bfloat16)

    return pl.pallas_call(
        functools.partial(_fused_kernel, heads=heads, out1=out1),
        grid=(2, n // BLK),
        in_specs=[
            full(n, ins),
            pl.BlockSpec((BLK, n),
                         lambda p, i: (jnp.where(p == 0, i, n // BLK - 1), 0)),
            full(ins, hidden),
            full(hidden, heads),
            full(hidden, heads),
            full(hidden, classes),
            full(classes, 8),
            full(classes, 8),
        ],
        out_specs=pl.BlockSpec((BLK, classes), lambda p, i: (i, 0)),
        out_shape=jax.ShapeDtypeStruct((n, classes), jnp.float32),
        scratch_shapes=[
            bf16_scr(n, n),
            f32_scr(n, hidden),
            bf16_scr(n, (out1 + 1) * heads),
            bf16_scr(n, heads),
            bf16_scr(n, heads),
            bf16_scr(heads, n),
            bf16_scr(heads, n),
            bf16_scr(n, classes + 1),
            bf16_scr(n, 8),
            bf16_scr(n, 8),
            bf16_scr(8, n),
            bf16_scr(8, n),
        ],
    )(x, adj, w1f, asrc1, adst1, W2[0], a2p_src, a2p_dst)


# speedup vs baseline: 1.1399x; 1.0526x over previous
import functools

import jax
import jax.numpy as jnp
from jax.experimental import pallas as pl
from jax.experimental.pallas import tpu as pltpu

BLK = 512
LEAK = 0.2


def _attention_factors(es, ed, dtype):
    ces = jnp.max(es, axis=0, keepdims=True)
    ced = jnp.max(ed, axis=0, keepdims=True)
    f1 = jnp.exp(es - ces)
    f2k = jnp.exp(LEAK * (es - ces) - (1.0 - LEAK) * (ces + ced))
    g1 = jnp.exp(ed - ced)
    g2 = jnp.exp(LEAK * (ed - ced))
    c = lambda v: v.astype(dtype)
    return c(f1), c(f2k), c(g1.T), c(g2.T)


def _block_weights(adj_blk, i, head, f1_ref, f2k_ref, g1t_ref, g2t_ref):
    rows = pl.ds(i * BLK, BLK)
    w1 = f1_ref[rows, head : head + 1] * g1t_ref[head : head + 1, :]
    w2 = f2k_ref[rows, head : head + 1] * g2t_ref[head : head + 1, :]
    return jnp.maximum(w1, w2) * adj_blk


def _fused_kernel(x_ref, adj_ref, w1_ref, asrc1_ref, adst1_ref, w2_ref,
                  asrc2_ref, adst2_ref, out_ref,
                  adj16_scr, h1_scr, hx1_scr, f11_scr, f21_scr, g11_scr,
                  g21_scr, hx2_scr, f12_scr, f22_scr, g12_scr, g22_scr,
                  *, heads, out1):
    i = pl.program_id(0)
    ext = out1 + 1
    classes = hx2_scr.shape[1] - 1
    nblk = adj16_scr.shape[0] // BLK

    @pl.when(i == 0)
    def _precompute1():
        h = jnp.dot(x_ref[...], w1_ref[...], preferred_element_type=jnp.float32)
        es = jnp.dot(h, asrc1_ref[...], preferred_element_type=jnp.float32)
        ed = jnp.dot(h, adst1_ref[...], preferred_element_type=jnp.float32)
        f1, f2k, g1t, g2t = _attention_factors(es, ed, jnp.bfloat16)
        f11_scr[...] = f1
        f21_scr[...] = f2k
        g11_scr[...] = g1t
        g21_scr[...] = g2t
        h16 = h.astype(jnp.bfloat16)
        for head in range(heads):
            hx1_scr[:, head * ext:head * ext + out1] = (
                h16[:, head * out1:(head + 1) * out1])
            hx1_scr[:, head * ext + out1:(head + 1) * ext] = jnp.ones(
                (h.shape[0], 1), jnp.bfloat16)

    @pl.when(i < nblk)
    def _layer1_block():
        adj_blk = adj_ref[...].astype(jnp.bfloat16)
        adj16_scr[pl.ds(i * BLK, BLK), :] = adj_blk
        for head in range(heads):
            p = _block_weights(adj_blk, i, head, f11_scr, f21_scr,
                               g11_scr, g21_scr)
            ne = jnp.dot(p, hx1_scr[:, head * ext:(head + 1) * ext],
                         preferred_element_type=jnp.float32)
            o = ne[:, :out1] / ne[:, out1:]
            h1_scr[pl.ds(i * BLK, BLK), head * out1:(head + 1) * out1] = (
                jnp.where(o > 0, o, jnp.exp(o) - 1.0))

    @pl.when(i == nblk)
    def _precompute2():
        h = jnp.dot(h1_scr[...], w2_ref[...],
                    preferred_element_type=jnp.float32)
        es = jnp.dot(h, asrc2_ref[...], preferred_element_type=jnp.float32)
        ed = jnp.dot(h, adst2_ref[...], preferred_element_type=jnp.float32)
        f1, f2k, g1t, g2t = _attention_factors(es, ed, jnp.bfloat16)
        f12_scr[...] = f1
        f22_scr[...] = f2k
        g12_scr[...] = g1t
        g22_scr[...] = g2t
        hx2_scr[:, :classes] = h.astype(jnp.bfloat16)
        hx2_scr[:, classes:] = jnp.ones((h.shape[0], 1), jnp.bfloat16)

    @pl.when(i == nblk)
    def _layer2_all():
        for c in range(nblk):
            adj_blk = adj16_scr[pl.ds(c * BLK, BLK), :]
            p = _block_weights(adj_blk, c, 0, f12_scr, f22_scr,
                               g12_scr, g22_scr)
            ne = jnp.dot(p, hx2_scr[...], preferred_element_type=jnp.float32)
            o = ne[:, :classes] / ne[:, classes:]
            m = jnp.max(o, axis=1, keepdims=True)
            lse = jnp.log(jnp.sum(jnp.exp(o - m), axis=1, keepdims=True)) + m
            out_ref[pl.ds(c * BLK, BLK), :] = o - lse


def kernel(x, adj, W1, a1_src, a1_dst, W2, a2_src, a2_dst):
    n, ins = x.shape
    heads, _, out1 = W1.shape
    classes = W2.shape[2]
    hidden = heads * out1

    w1f = jnp.transpose(W1, (1, 0, 2)).reshape(ins, hidden)
    eye = jnp.eye(heads, dtype=x.dtype)
    asrc1 = (a1_src[:, :, None] * eye[:, None, :]).reshape(hidden, heads)
    adst1 = (a1_dst[:, :, None] * eye[:, None, :]).reshape(hidden, heads)

    pad = jnp.zeros((classes, 7), x.dtype)
    a2p_src = jnp.concatenate([a2_src.reshape(classes, 1), pad], axis=1)
    a2p_dst = jnp.concatenate([a2_dst.reshape(classes, 1), pad], axis=1)

    nblk = n // BLK
    full = lambda *dims: pl.BlockSpec(dims, lambda i: (0,) * len(dims))
    f32_scr = lambda r, c: pltpu.VMEM((r, c), jnp.float32)
    bf16_scr = lambda r, c: pltpu.VMEM((r, c), jnp.bfloat16)

    return pl.pallas_call(
        functools.partial(_fused_kernel, heads=heads, out1=out1),
        grid=(nblk + 1,),
        in_specs=[
            full(n, ins),
            pl.BlockSpec((BLK, n),
                         lambda i: (jnp.minimum(i, n // BLK - 1), 0)),
            full(ins, hidden),
            full(hidden, heads),
            full(hidden, heads),
            full(hidden, classes),
            full(classes, 8),
            full(classes, 8),
        ],
        out_specs=pl.BlockSpec((n, classes), lambda i: (0, 0)),
        out_shape=jax.ShapeDtypeStruct((n, classes), jnp.float32),
        scratch_shapes=[
            bf16_scr(n, n),
            f32_scr(n, hidden),
            bf16_scr(n, (out1 + 1) * heads),
            bf16_scr(n, heads),
            bf16_scr(n, heads),
            bf16_scr(heads, n),
            bf16_scr(heads, n),
            bf16_scr(n, classes + 1),
            bf16_scr(n, 8),
            bf16_scr(n, 8),
            bf16_scr(8, n),
            bf16_scr(8, n),
        ],
    )(x, adj, w1f, asrc1, adst1, W2[0], a2p_src, a2p_dst)
